# baseline (speedup 1.0000x reference)
import jax
import jax.numpy as jnp
from jax import lax
from jax.experimental import pallas as pl
from jax.experimental.pallas import tpu as pltpu
from jax.experimental.pallas import tpu_sc as plsc

_B, _T, _V, _L = 4, 256, 16, 8
_N = 40
_ROW = _T * _N
_NC, _NS = 2, 16
_NW = _NC * _NS
_PAIRS_PER_W = (_B * _V) // _NW
_MROW = 32


def _sc_body(srl_hbm, vn_hbm, meta_hbm, out_hbm, meta_v, slab_s, slab_v, res_v):
    wid = lax.axis_index("s") * _NC + lax.axis_index("c")
    iota = lax.iota(jnp.int32, 16)

    total = jnp.zeros((16,), jnp.float32)
    for j in range(_PAIRS_PER_W):
        pair = wid * _PAIRS_PER_W + j
        b = pair // _V
        v = pair - b * _V
        pltpu.sync_copy(meta_hbm.at[pair], meta_v)
        m0 = meta_v[pl.ds(0, 16)]
        m1 = meta_v[pl.ds(16, 16)]

        row = b * _T + m0[0]
        pltpu.sync_copy(srl_hbm.at[row], slab_s)
        pltpu.sync_copy(vn_hbm.at[row], slab_v)

        v_ok = v < m0[2]
        sll = m0[1]
        oln = jnp.where(b == 0, m0[3],
                        jnp.where(b == 1, m0[4],
                                  jnp.where(b == 2, m0[5], m0[6])))
        nrm = m0[3] + m0[4] + m0[5] + m0[6]
        inv_vec = 1.0 / jnp.full((16,), nrm.astype(jnp.float32))
        nchunks = (oln + 15) // 16

        for l in range(_L):
            r = m0[8 + l]
            a = m1[l]
            coef = ((l < sll) & v_ok).astype(jnp.float32)
            rmask = (r != 0).astype(jnp.float32)
            amask = (a != 0).astype(jnp.float32)

            def chunk_body(c, acc, r=r, a=a, rmask=rmask, amask=amask,
                           oln=oln):
                t = c * 16 + iota
                x = plsc.load_gather(slab_s, [t * _N + r]) * rmask
                y = plsc.load_gather(slab_v, [t * _N + a]) * amask
                tm = (t < oln).astype(jnp.float32)
                return acc + jnp.abs(x - y) * tm

            acc_l = lax.fori_loop(0, nchunks, chunk_body,
                                  jnp.zeros((16,), jnp.float32))
            total = total + acc_l * coef

    res_v[...] = total * inv_vec
    pltpu.sync_copy(res_v, out_hbm.at[wid])


def kernel(log_srl, log_vn, v_label, v_l, orig_l, semlink, semlink_l):
    srl2 = log_srl.reshape(_B * _T, _ROW)
    vn2 = log_vn.reshape(_B * _T, _ROW)

    bv = _B * _V
    meta = jnp.zeros((bv, _MROW), jnp.int32)
    meta = meta.at[:, 0].set(v_label.astype(jnp.int32).reshape(bv))
    meta = meta.at[:, 1].set(semlink_l.astype(jnp.int32).reshape(bv))
    meta = meta.at[:, 2].set(
        jnp.repeat(v_l.astype(jnp.int32), _V, total_repeat_length=bv))
    meta = meta.at[:, 3:7].set(
        jnp.broadcast_to(orig_l.astype(jnp.int32), (bv, _B)))
    meta = meta.at[:, 8:16].set(semlink[:, :, 0, :].astype(jnp.int32)
                                .reshape(bv, _L))
    meta = meta.at[:, 16:24].set(semlink[:, :, 1, :].astype(jnp.int32)
                                 .reshape(bv, _L))

    sc_call = pl.kernel(
        _sc_body,
        out_type=jax.ShapeDtypeStruct((_NW, 16), jnp.float32),
        mesh=plsc.VectorSubcoreMesh(core_axis_name="c", subcore_axis_name="s"),
        scratch_types=[
            pltpu.VMEM((_MROW,), jnp.int32),
            pltpu.VMEM((_ROW,), jnp.float32),
            pltpu.VMEM((_ROW,), jnp.float32),
            pltpu.VMEM((16,), jnp.float32),
        ],
        compiler_params=pltpu.CompilerParams(needs_layout_passes=False),
    )
    partials = sc_call(srl2, vn2, meta)
    return jnp.sum(partials)


# speedup vs baseline: 5.5099x; 5.5099x over previous
import jax
import jax.numpy as jnp
from jax import lax
from jax.experimental import pallas as pl
from jax.experimental.pallas import tpu as pltpu
from jax.experimental.pallas import tpu_sc as plsc

_B, _T, _V, _L = 4, 256, 16, 8
_N = 40
_NC, _NS = 2, 16
_NW = _NC * _NS
_PAIRS_PER_W = (_B * _V) // _NW
_MROW = 32


def _sc_body(srl_hbm, vn_hbm, meta_hbm, out_hbm, meta_v, rows_v, res_v, sem):
    wid = lax.axis_index("s") * _NC + lax.axis_index("c")
    iota = lax.iota(jnp.int32, 16)

    total = jnp.zeros((16,), jnp.float32)
    for j in range(_PAIRS_PER_W):
        pair = wid * _PAIRS_PER_W + j
        b = pair // _V
        v = pair - b * _V
        pltpu.sync_copy(meta_hbm.at[pair], meta_v)
        m0 = meta_v[pl.ds(0, 16)]
        m1 = meta_v[pl.ds(16, 16)]

        vlab = m0[0]
        copies = []
        for l in range(_L):
            copies.append(pltpu.async_copy(
                srl_hbm.at[b, vlab, m0[8 + l]],
                rows_v.at[pl.ds(l * _T, _T)], sem))
            copies.append(pltpu.async_copy(
                vn_hbm.at[b, vlab, m1[l]],
                rows_v.at[pl.ds((_L + l) * _T, _T)], sem))
        for c in copies:
            c.wait()

        v_ok = v < m0[2]
        sll = m0[1]
        oln = jnp.where(b == 0, m0[3],
                        jnp.where(b == 1, m0[4],
                                  jnp.where(b == 2, m0[5], m0[6])))
        nrm = m0[3] + m0[4] + m0[5] + m0[6]
        inv_vec = 1.0 / jnp.full((16,), nrm.astype(jnp.float32))
        nchunks = (oln + 15) // 16

        for l in range(_L):
            r = m0[8 + l]
            a = m1[l]
            coef = ((l < sll) & v_ok).astype(jnp.float32)
            rmask = (r != 0).astype(jnp.float32)
            amask = (a != 0).astype(jnp.float32)

            def chunk_body(c, acc, l=l, rmask=rmask, amask=amask, oln=oln):
                t = c * 16 + iota
                x = rows_v[pl.ds(l * _T + c * 16, 16)] * rmask
                y = rows_v[pl.ds((_L + l) * _T + c * 16, 16)] * amask
                tm = (t < oln).astype(jnp.float32)
                return acc + jnp.abs(x - y) * tm

            acc_l = lax.fori_loop(0, nchunks, chunk_body,
                                  jnp.zeros((16,), jnp.float32))
            total = total + acc_l * coef

    res_v[...] = total * inv_vec
    pltpu.sync_copy(res_v, out_hbm.at[wid])


def kernel(log_srl, log_vn, v_label, v_l, orig_l, semlink, semlink_l):
    srl_t = jnp.transpose(log_srl, (0, 1, 3, 2))
    vn_t = jnp.transpose(log_vn, (0, 1, 3, 2))

    bv = _B * _V
    meta = jnp.zeros((bv, _MROW), jnp.int32)
    meta = meta.at[:, 0].set(v_label.astype(jnp.int32).reshape(bv))
    meta = meta.at[:, 1].set(semlink_l.astype(jnp.int32).reshape(bv))
    meta = meta.at[:, 2].set(
        jnp.repeat(v_l.astype(jnp.int32), _V, total_repeat_length=bv))
    meta = meta.at[:, 3:7].set(
        jnp.broadcast_to(orig_l.astype(jnp.int32), (bv, _B)))
    meta = meta.at[:, 8:16].set(semlink[:, :, 0, :].astype(jnp.int32)
                                .reshape(bv, _L))
    meta = meta.at[:, 16:24].set(semlink[:, :, 1, :].astype(jnp.int32)
                                 .reshape(bv, _L))

    sc_call = pl.kernel(
        _sc_body,
        out_type=jax.ShapeDtypeStruct((_NW, 16), jnp.float32),
        mesh=plsc.VectorSubcoreMesh(core_axis_name="c", subcore_axis_name="s"),
        scratch_types=[
            pltpu.VMEM((_MROW,), jnp.int32),
            pltpu.VMEM((2 * _L * _T,), jnp.float32),
            pltpu.VMEM((16,), jnp.float32),
            pltpu.SemaphoreType.DMA,
        ],
        compiler_params=pltpu.CompilerParams(
            needs_layout_passes=False,
            use_tc_tiling_on_sc=True,
        ),
    )
    partials = sc_call(srl_t, vn_t, meta)
    return jnp.sum(partials)


# speedup vs baseline: 8.2497x; 1.4973x over previous
import jax
import jax.numpy as jnp
from jax import lax
from jax.experimental import pallas as pl
from jax.experimental.pallas import tpu as pltpu
from jax.experimental.pallas import tpu_sc as plsc

_B, _T, _V, _L = 4, 256, 16, 8
_N = 40
_NC, _NS = 2, 16
_NW = _NC * _NS
_PAIRS_PER_W = (_B * _V) // _NW
_MROW = 32


def _sc_body(srl_hbm, vn_hbm, meta_hbm, out_hbm, meta_v, rows_v, res_v, sem):
    wid = lax.axis_index("s") * _NC + lax.axis_index("c")
    iota = lax.iota(jnp.int32, 16)

    total = jnp.zeros((16,), jnp.float32)
    for j in range(_PAIRS_PER_W):
        pair = wid * _PAIRS_PER_W + j
        b = pair // _V
        v = pair - b * _V
        pltpu.sync_copy(meta_hbm.at[pair], meta_v)
        m0 = meta_v[pl.ds(0, 16)]
        m1 = meta_v[pl.ds(16, 16)]

        vlab = m0[0]
        copies = []
        for l in range(_L):
            copies.append(pltpu.async_copy(
                srl_hbm.at[b, vlab, m0[8 + l]],
                rows_v.at[pl.ds(l * _T, _T)], sem))
            copies.append(pltpu.async_copy(
                vn_hbm.at[b, vlab, m1[l]],
                rows_v.at[pl.ds((_L + l) * _T, _T)], sem))
        for c in copies:
            c.wait()

        v_ok = v < m0[2]
        sll = m0[1]
        oln = jnp.where(b == 0, m0[3],
                        jnp.where(b == 1, m0[4],
                                  jnp.where(b == 2, m0[5], m0[6])))
        nrm = m0[3] + m0[4] + m0[5] + m0[6]
        inv_vec = 1.0 / jnp.full((16,), nrm.astype(jnp.float32))
        nchunks = (oln + 15) // 16

        for l in range(_L):
            r = m0[8 + l]
            a = m1[l]
            coef = ((l < sll) & v_ok).astype(jnp.float32)
            rmask = (r != 0).astype(jnp.float32)
            amask = (a != 0).astype(jnp.float32)

            def chunk_body(c, acc, l=l, rmask=rmask, amask=amask, oln=oln):
                t = c * 16 + iota
                x = rows_v[pl.ds(l * _T + c * 16, 16)] * rmask
                y = rows_v[pl.ds((_L + l) * _T + c * 16, 16)] * amask
                tm = (t < oln).astype(jnp.float32)
                return acc + jnp.abs(x - y) * tm

            acc_l = lax.fori_loop(0, nchunks, chunk_body,
                                  jnp.zeros((16,), jnp.float32))
            total = total + acc_l * coef

    res_v[...] = total * inv_vec
    pltpu.sync_copy(res_v, out_hbm.at[wid])


def kernel(log_srl, log_vn, v_label, v_l, orig_l, semlink, semlink_l):
    srl_t = jnp.transpose(log_srl, (0, 1, 3, 2))
    vn_t = jnp.transpose(log_vn, (0, 1, 3, 2))

    bv = _B * _V
    meta = jnp.concatenate([
        v_label.astype(jnp.int32).reshape(bv, 1),
        semlink_l.astype(jnp.int32).reshape(bv, 1),
        jnp.broadcast_to(v_l.astype(jnp.int32)[:, None, None],
                         (_B, _V, 1)).reshape(bv, 1),
        jnp.broadcast_to(orig_l.astype(jnp.int32)[None, :], (bv, _B)),
        jnp.zeros((bv, 1), jnp.int32),
        semlink.astype(jnp.int32).reshape(bv, 2 * _L),
        jnp.zeros((bv, _MROW - 24), jnp.int32),
    ], axis=1)

    sc_call = pl.kernel(
        _sc_body,
        out_type=jax.ShapeDtypeStruct((_NW, 16), jnp.float32),
        mesh=plsc.VectorSubcoreMesh(core_axis_name="c", subcore_axis_name="s"),
        scratch_types=[
            pltpu.VMEM((_MROW,), jnp.int32),
            pltpu.VMEM((2 * _L * _T,), jnp.float32),
            pltpu.VMEM((16,), jnp.float32),
            pltpu.SemaphoreType.DMA,
        ],
        compiler_params=pltpu.CompilerParams(
            needs_layout_passes=False,
            use_tc_tiling_on_sc=True,
        ),
    )
    partials = sc_call(srl_t, vn_t, meta)
    return jnp.sum(partials)
